# baseline (speedup 1.0000x reference)
import functools

import jax
import jax.numpy as jnp
from jax import lax
from jax.experimental import pallas as pl
from jax.experimental.pallas import tpu as pltpu
from jax.experimental.pallas import tpu_sc as plsc


def _sc_partial_sumsq(logits, label, table):
    B, P = logits.shape
    L = table.shape[0]
    info = plsc.get_sparse_core_info()
    NC, NS, LN = info.num_cores, info.num_subcores, info.num_lanes
    NW = NC * NS
    rows_per = B // NW
    CH = 256
    nch = rows_per // CH
    NB = nch

    mesh = plsc.VectorSubcoreMesh(core_axis_name="c", subcore_axis_name="s")

    @functools.partial(
        pl.kernel,
        mesh=mesh,
        out_type=jax.ShapeDtypeStruct((B,), jnp.float32),
        compiler_params=pltpu.CompilerParams(needs_layout_passes=False),
        scratch_types=[
            pltpu.VMEM((rows_per,), jnp.int32),
            pltpu.VMEM((L, P), jnp.float32),
            pltpu.VMEM((CH, P), jnp.float32),
            pltpu.VMEM((CH, P), jnp.float32),
            pltpu.VMEM((CH,), jnp.float32),
            pltpu.VMEM((CH,), jnp.float32),
            pltpu.SemaphoreType.DMA,
            pltpu.SemaphoreType.DMA,
            pltpu.SemaphoreType.DMA,
            pltpu.SemaphoreType.DMA,
        ],
    )
    def k(logits_hbm, label_hbm, table_hbm, out_hbm, idx_v, tab_v,
          log_v0, log_v1, out_v0, out_v1, lsem0, lsem1, osem0, osem1):
        wid = lax.axis_index("s") * NC + lax.axis_index("c")
        base = wid * rows_per
        logs = (log_v0, log_v1)
        outs = (out_v0, out_v1)
        lsems = (lsem0, lsem1)
        osems = (osem0, osem1)

        icp = pltpu.async_copy(
            label_hbm.at[pl.ds(base, rows_per)], idx_v, osems[0])
        tcp = pltpu.async_copy(table_hbm, tab_v, osems[1])
        lcp = [pltpu.async_copy(
            logits_hbm.at[pl.ds(base + ci * CH, CH)], logs[ci], lsems[ci])
            for ci in range(nch)]
        icp.wait()
        tcp.wait()

        iota = lax.iota(jnp.int32, LN)
        cols = [iota + j * LN for j in range(P // LN)]
        last = jnp.full((LN,), LN - 1, jnp.int32)
        z16 = jnp.zeros((LN,), jnp.float32)

        ocp = [None] * nch
        for ci in range(nch):
            b = ci % NB
            lcp[ci].wait()
            log_v = logs[b]
            out_v = outs[b]
            coff = ci * CH

            @plsc.parallel_loop(0, CH, 1, unroll=4, carry=z16)
            def row(r, sel):
                labv = plsc.load_gather(
                    idx_v, [jnp.full((LN,), coff + r, jnp.int32)])
                acc = jnp.zeros((LN,), jnp.float32)
                for j in range(P // LN):
                    t = plsc.load_gather(tab_v, [labv, cols[j]])
                    d = log_v[r, pl.ds(j * LN, LN)] - t
                    acc = acc + d * d
                cs = jnp.cumsum(acc)
                tot = jnp.take(cs, last)
                lane = r % LN
                sel = jnp.where(iota == lane, tot, sel)
                done = lane == LN - 1

                @pl.when(done)
                def _():
                    out_v[pl.ds((r // LN) * LN, LN)] = sel

                return jnp.where(done, z16, sel)

            ocp[ci] = pltpu.async_copy(
                out_v, out_hbm.at[pl.ds(base + ci * CH, CH)], osems[b])
        for ci in range(nch):
            ocp[ci].wait()

    return k(logits, label, table)


def _tc_finish(x, B):
    def body(x_ref, o_ref):
        o_ref[0, 0] = jnp.sum(jnp.sqrt(x_ref[...])) / B

    out = pl.pallas_call(
        body,
        out_shape=jax.ShapeDtypeStruct((1, 1), jnp.float32),
        out_specs=pl.BlockSpec(memory_space=pltpu.SMEM),
    )(x)
    return out[0, 0]


def kernel(logits, label, mean_expand):
    label = label.astype(jnp.int32)
    partial = _sc_partial_sumsq(logits, label, mean_expand)
    return _tc_finish(partial, logits.shape[0])


# speedup vs baseline: 1.0282x; 1.0282x over previous
import functools

import jax
import jax.numpy as jnp
from jax import lax
from jax.experimental import pallas as pl
from jax.experimental.pallas import tpu as pltpu
from jax.experimental.pallas import tpu_sc as plsc


def _sc_partial_sumsq(logits, label, table):
    B, P = logits.shape
    L = table.shape[0]
    info = plsc.get_sparse_core_info()
    NC, NS, LN = info.num_cores, info.num_subcores, info.num_lanes
    NW = NC * NS
    rows_per = B // NW
    CH = 256
    nch = rows_per // CH
    NB = nch

    mesh = plsc.VectorSubcoreMesh(core_axis_name="c", subcore_axis_name="s")

    @functools.partial(
        pl.kernel,
        mesh=mesh,
        out_type=jax.ShapeDtypeStruct((B,), jnp.float32),
        compiler_params=pltpu.CompilerParams(needs_layout_passes=False),
        scratch_types=[
            pltpu.VMEM((rows_per,), jnp.int32),
            pltpu.VMEM((L, P), jnp.float32),
            pltpu.VMEM((CH, P), jnp.float32),
            pltpu.VMEM((CH, P), jnp.float32),
            pltpu.VMEM((CH,), jnp.float32),
            pltpu.VMEM((CH,), jnp.float32),
            pltpu.SemaphoreType.DMA,
            pltpu.SemaphoreType.DMA,
            pltpu.SemaphoreType.DMA,
            pltpu.SemaphoreType.DMA,
        ],
    )
    def k(logits_hbm, label_hbm, table_hbm, out_hbm, idx_v, tab_v,
          log_v0, log_v1, out_v0, out_v1, lsem0, lsem1, osem0, osem1):
        wid = lax.axis_index("s") * NC + lax.axis_index("c")
        base = wid * rows_per
        logs = (log_v0, log_v1)
        outs = (out_v0, out_v1)
        lsems = (lsem0, lsem1)
        osems = (osem0, osem1)

        lcp = [pltpu.async_copy(
            logits_hbm.at[pl.ds(base + ci * CH, CH)], logs[ci], lsems[ci])
            for ci in range(nch)]
        pltpu.sync_copy(label_hbm.at[pl.ds(base, rows_per)], idx_v)
        pltpu.sync_copy(table_hbm, tab_v)

        iota = lax.iota(jnp.int32, LN)
        cols = [iota + j * LN for j in range(P // LN)]
        last = jnp.full((LN,), LN - 1, jnp.int32)
        z16 = jnp.zeros((LN,), jnp.float32)

        ocp = [None] * nch
        for ci in range(nch):
            b = ci % NB
            lcp[ci].wait()
            log_v = logs[b]
            out_v = outs[b]
            coff = ci * CH

            @plsc.parallel_loop(0, CH, 1, unroll=2, carry=z16)
            def row(r, sel):
                labv = plsc.load_gather(
                    idx_v, [jnp.full((LN,), coff + r, jnp.int32)])
                acc = jnp.zeros((LN,), jnp.float32)
                for j in range(P // LN):
                    t = plsc.load_gather(tab_v, [labv, cols[j]])
                    d = log_v[r, pl.ds(j * LN, LN)] - t
                    acc = acc + d * d
                cs = jnp.cumsum(acc)
                tot = jnp.take(cs, last)
                lane = r % LN
                sel = jnp.where(iota == lane, tot, sel)
                done = lane == LN - 1

                @pl.when(done)
                def _():
                    out_v[pl.ds((r // LN) * LN, LN)] = sel

                return jnp.where(done, z16, sel)

            ocp[ci] = pltpu.async_copy(
                out_v, out_hbm.at[pl.ds(base + ci * CH, CH)], osems[b])
        for ci in range(nch):
            ocp[ci].wait()

    return k(logits, label, table)


def _tc_finish(x, B):
    def body(x_ref, o_ref):
        o_ref[0, 0] = jnp.sum(jnp.sqrt(x_ref[...])) / B

    out = pl.pallas_call(
        body,
        out_shape=jax.ShapeDtypeStruct((1, 1), jnp.float32),
        out_specs=pl.BlockSpec(memory_space=pltpu.SMEM),
    )(x)
    return out[0, 0]


def kernel(logits, label, mean_expand):
    label = label.astype(jnp.int32)
    partial = _sc_partial_sumsq(logits, label, mean_expand)
    return _tc_finish(partial, logits.shape[0])


# speedup vs baseline: 1.0304x; 1.0022x over previous
import functools

import jax
import jax.numpy as jnp
from jax import lax
from jax.experimental import pallas as pl
from jax.experimental.pallas import tpu as pltpu
from jax.experimental.pallas import tpu_sc as plsc


def _sc_partial_sumsq(logits, label, table):
    B, P = logits.shape
    L = table.shape[0]
    info = plsc.get_sparse_core_info()
    NC, NS, LN = info.num_cores, info.num_subcores, info.num_lanes
    NW = NC * NS
    rows_per = B // NW
    CH = 256
    nch = rows_per // CH
    NB = nch

    mesh = plsc.VectorSubcoreMesh(core_axis_name="c", subcore_axis_name="s")

    @functools.partial(
        pl.kernel,
        mesh=mesh,
        out_type=jax.ShapeDtypeStruct((B,), jnp.float32),
        compiler_params=pltpu.CompilerParams(needs_layout_passes=False),
        scratch_types=[
            pltpu.VMEM((rows_per,), jnp.int32),
            pltpu.VMEM((L, P), jnp.float32),
            pltpu.VMEM((CH, P), jnp.float32),
            pltpu.VMEM((CH, P), jnp.float32),
            pltpu.VMEM((CH,), jnp.float32),
            pltpu.VMEM((CH,), jnp.float32),
            pltpu.SemaphoreType.DMA,
            pltpu.SemaphoreType.DMA,
            pltpu.SemaphoreType.DMA,
            pltpu.SemaphoreType.DMA,
        ],
    )
    def k(logits_hbm, label_hbm, table_hbm, out_hbm, idx_v, tab_v,
          log_v0, log_v1, out_v0, out_v1, lsem0, lsem1, osem0, osem1):
        wid = lax.axis_index("s") * NC + lax.axis_index("c")
        base = wid * rows_per
        logs = (log_v0, log_v1)
        outs = (out_v0, out_v1)
        lsems = (lsem0, lsem1)
        osems = (osem0, osem1)

        lcp = [pltpu.async_copy(
            logits_hbm.at[pl.ds(base + ci * CH, CH)], logs[ci], lsems[ci])
            for ci in range(nch)]
        pltpu.sync_copy(label_hbm.at[pl.ds(base, rows_per)], idx_v)
        pltpu.sync_copy(table_hbm, tab_v)

        iota = lax.iota(jnp.int32, LN)
        cols = [iota + j * LN for j in range(P // LN)]
        last = jnp.full((LN,), LN - 1, jnp.int32)
        z16 = jnp.zeros((LN,), jnp.float32)

        ocp = [None] * nch
        for ci in range(nch):
            b = ci % NB
            lcp[ci].wait()
            log_v = logs[b]
            out_v = outs[b]
            coff = ci * CH

            @plsc.parallel_loop(0, CH, 1, unroll=1, carry=z16)
            def row(r, sel):
                labv = plsc.load_gather(
                    idx_v, [jnp.full((LN,), coff + r, jnp.int32)])
                acc = jnp.zeros((LN,), jnp.float32)
                for j in range(P // LN):
                    t = plsc.load_gather(tab_v, [labv, cols[j]])
                    d = log_v[r, pl.ds(j * LN, LN)] - t
                    acc = acc + d * d
                cs = jnp.cumsum(acc)
                tot = jnp.take(cs, last)
                lane = r % LN
                sel = jnp.where(iota == lane, tot, sel)
                done = lane == LN - 1

                @pl.when(done)
                def _():
                    out_v[pl.ds((r // LN) * LN, LN)] = sel

                return jnp.where(done, z16, sel)

            ocp[ci] = pltpu.async_copy(
                out_v, out_hbm.at[pl.ds(base + ci * CH, CH)], osems[b])
        for ci in range(nch):
            ocp[ci].wait()

    return k(logits, label, table)


def _tc_finish(x, B):
    def body(x_ref, o_ref):
        o_ref[0, 0] = jnp.sum(jnp.sqrt(x_ref[...])) / B

    out = pl.pallas_call(
        body,
        out_shape=jax.ShapeDtypeStruct((1, 1), jnp.float32),
        out_specs=pl.BlockSpec(memory_space=pltpu.SMEM),
    )(x)
    return out[0, 0]


def kernel(logits, label, mean_expand):
    label = label.astype(jnp.int32)
    partial = _sc_partial_sumsq(logits, label, mean_expand)
    return _tc_finish(partial, logits.shape[0])


# speedup vs baseline: 1.0728x; 1.0412x over previous
import functools

import jax
import jax.numpy as jnp
from jax import lax
from jax.experimental import pallas as pl
from jax.experimental.pallas import tpu as pltpu
from jax.experimental.pallas import tpu_sc as plsc


def _sc_partial_sumsq(logits, label, table, row0, bsc):
    _, P = logits.shape
    L = table.shape[0]
    info = plsc.get_sparse_core_info()
    NC, NS, LN = info.num_cores, info.num_subcores, info.num_lanes
    NW = NC * NS
    rows_per = bsc // NW
    CH = rows_per // 2
    nch = rows_per // CH
    NB = nch

    mesh = plsc.VectorSubcoreMesh(core_axis_name="c", subcore_axis_name="s")

    @functools.partial(
        pl.kernel,
        mesh=mesh,
        out_type=jax.ShapeDtypeStruct((bsc,), jnp.float32),
        compiler_params=pltpu.CompilerParams(needs_layout_passes=False),
        scratch_types=[
            pltpu.VMEM((rows_per,), jnp.int32),
            pltpu.VMEM((L, P), jnp.float32),
            pltpu.VMEM((CH, P), jnp.float32),
            pltpu.VMEM((CH, P), jnp.float32),
            pltpu.VMEM((CH,), jnp.float32),
            pltpu.VMEM((CH,), jnp.float32),
            pltpu.SemaphoreType.DMA,
            pltpu.SemaphoreType.DMA,
            pltpu.SemaphoreType.DMA,
            pltpu.SemaphoreType.DMA,
        ],
    )
    def k(logits_hbm, label_hbm, table_hbm, out_hbm, idx_v, tab_v,
          log_v0, log_v1, out_v0, out_v1, lsem0, lsem1, osem0, osem1):
        wid = lax.axis_index("s") * NC + lax.axis_index("c")
        obase = pl.multiple_of(wid * rows_per, 8)
        base = row0 + obase
        logs = (log_v0, log_v1)
        outs = (out_v0, out_v1)
        lsems = (lsem0, lsem1)
        osems = (osem0, osem1)

        lcp = [pltpu.async_copy(
            logits_hbm.at[pl.ds(base + ci * CH, CH)], logs[ci], lsems[ci])
            for ci in range(nch)]
        pltpu.sync_copy(label_hbm.at[pl.ds(base, rows_per)], idx_v)
        pltpu.sync_copy(table_hbm, tab_v)

        iota = lax.iota(jnp.int32, LN)
        cols = [iota + j * LN for j in range(P // LN)]
        last = jnp.full((LN,), LN - 1, jnp.int32)
        z16 = jnp.zeros((LN,), jnp.float32)

        ocp = [None] * nch
        for ci in range(nch):
            b = ci % NB
            lcp[ci].wait()
            log_v = logs[b]
            out_v = outs[b]
            coff = ci * CH

            @plsc.parallel_loop(0, CH, 1, unroll=1, carry=z16)
            def row(r, sel):
                labv = plsc.load_gather(
                    idx_v, [jnp.full((LN,), coff + r, jnp.int32)])
                acc = jnp.zeros((LN,), jnp.float32)
                for j in range(P // LN):
                    t = plsc.load_gather(tab_v, [labv, cols[j]])
                    d = log_v[r, pl.ds(j * LN, LN)] - t
                    acc = acc + d * d
                cs = jnp.cumsum(acc)
                tot = jnp.take(cs, last)
                lane = r % LN
                sel = jnp.where(iota == lane, tot, sel)
                done = lane == LN - 1

                @pl.when(done)
                def _():
                    out_v[pl.ds((r // LN) * LN, LN)] = sel

                return jnp.where(done, z16, sel)

            ocp[ci] = pltpu.async_copy(
                out_v, out_hbm.at[pl.ds(obase + ci * CH, CH)], osems[b])
        for ci in range(nch):
            ocp[ci].wait()

    return k(logits, label, table)


def _tc_head(logits, label, table, btc):
    L, P = table.shape

    def body(x_ref, lab_ref, tab_ref, o_ref):
        lab = lab_ref[...]
        one_hot = (lab[:, None] ==
                   lax.broadcasted_iota(jnp.int32, (btc, L), 1))
        gathered = jnp.dot(one_hot.astype(jnp.float32), tab_ref[...],
                           preferred_element_type=jnp.float32)
        diff = x_ref[...] - gathered
        rowsum = jnp.sum(diff * diff, axis=1)
        o_ref[0, 0] = jnp.sum(jnp.sqrt(rowsum))

    out = pl.pallas_call(
        body,
        grid=(1,),
        in_specs=[
            pl.BlockSpec((btc, P), lambda i: (0, 0)),
            pl.BlockSpec((btc,), lambda i: (0,)),
            pl.BlockSpec((L, P), lambda i: (0, 0)),
        ],
        out_specs=pl.BlockSpec(memory_space=pltpu.SMEM),
        out_shape=jax.ShapeDtypeStruct((1, 1), jnp.float32),
    )(logits, label, table)
    return out


def _tc_finish(x, head_sum, B):
    def body(x_ref, h_ref, o_ref):
        o_ref[0, 0] = (jnp.sum(jnp.sqrt(x_ref[...])) + h_ref[0, 0]) / B

    out = pl.pallas_call(
        body,
        in_specs=[
            pl.BlockSpec(memory_space=pltpu.VMEM),
            pl.BlockSpec(memory_space=pltpu.SMEM),
        ],
        out_shape=jax.ShapeDtypeStruct((1, 1), jnp.float32),
        out_specs=pl.BlockSpec(memory_space=pltpu.SMEM),
    )(x, head_sum)
    return out[0, 0]


def kernel(logits, label, mean_expand):
    B = logits.shape[0]
    btc = B // 4
    bsc = B - btc
    label = label.astype(jnp.int32)
    head = _tc_head(logits, label, mean_expand, btc)
    part = _sc_partial_sumsq(logits, label, mean_expand, btc, bsc)
    return _tc_finish(part, head, B)


# speedup vs baseline: 1.1383x; 1.0610x over previous
import functools

import jax
import jax.numpy as jnp
from jax import lax
from jax.experimental import pallas as pl
from jax.experimental.pallas import tpu as pltpu
from jax.experimental.pallas import tpu_sc as plsc


def _sc_partial_sumsq(logits, label, table, row0, bsc):
    _, P = logits.shape
    L = table.shape[0]
    info = plsc.get_sparse_core_info()
    NC, NS, LN = info.num_cores, info.num_subcores, info.num_lanes
    NW = NC * NS
    rows_per = bsc // NW
    CH = rows_per // 2
    nch = rows_per // CH
    NB = nch

    mesh = plsc.VectorSubcoreMesh(core_axis_name="c", subcore_axis_name="s")

    @functools.partial(
        pl.kernel,
        mesh=mesh,
        out_type=jax.ShapeDtypeStruct((bsc,), jnp.float32),
        compiler_params=pltpu.CompilerParams(needs_layout_passes=False),
        scratch_types=[
            pltpu.VMEM((rows_per,), jnp.int32),
            pltpu.VMEM((L, P), jnp.float32),
            pltpu.VMEM((CH, P), jnp.float32),
            pltpu.VMEM((CH, P), jnp.float32),
            pltpu.VMEM((CH,), jnp.float32),
            pltpu.VMEM((CH,), jnp.float32),
            pltpu.SemaphoreType.DMA,
            pltpu.SemaphoreType.DMA,
            pltpu.SemaphoreType.DMA,
            pltpu.SemaphoreType.DMA,
        ],
    )
    def k(logits_hbm, label_hbm, table_hbm, out_hbm, idx_v, tab_v,
          log_v0, log_v1, out_v0, out_v1, lsem0, lsem1, osem0, osem1):
        wid = lax.axis_index("s") * NC + lax.axis_index("c")
        obase = pl.multiple_of(wid * rows_per, 8)
        base = row0 + obase
        logs = (log_v0, log_v1)
        outs = (out_v0, out_v1)
        lsems = (lsem0, lsem1)
        osems = (osem0, osem1)

        lcp = [pltpu.async_copy(
            logits_hbm.at[pl.ds(base + ci * CH, CH)], logs[ci], lsems[ci])
            for ci in range(nch)]
        pltpu.sync_copy(label_hbm.at[pl.ds(base, rows_per)], idx_v)
        pltpu.sync_copy(table_hbm, tab_v)

        iota = lax.iota(jnp.int32, LN)
        cols = [iota + j * LN for j in range(P // LN)]
        last = jnp.full((LN,), LN - 1, jnp.int32)
        z16 = jnp.zeros((LN,), jnp.float32)

        ocp = [None] * nch
        for ci in range(nch):
            b = ci % NB
            lcp[ci].wait()
            log_v = logs[b]
            out_v = outs[b]
            coff = ci * CH

            @plsc.parallel_loop(0, CH, 1, unroll=1, carry=z16)
            def row(r, sel):
                labv = plsc.load_gather(
                    idx_v, [jnp.full((LN,), coff + r, jnp.int32)])
                acc = jnp.zeros((LN,), jnp.float32)
                for j in range(P // LN):
                    t = plsc.load_gather(tab_v, [labv, cols[j]])
                    d = log_v[r, pl.ds(j * LN, LN)] - t
                    acc = acc + d * d
                cs = jnp.cumsum(acc)
                tot = jnp.take(cs, last)
                lane = r % LN
                sel = jnp.where(iota == lane, tot, sel)
                done = lane == LN - 1

                @pl.when(done)
                def _():
                    out_v[pl.ds((r // LN) * LN, LN)] = sel

                return jnp.where(done, z16, sel)

            ocp[ci] = pltpu.async_copy(
                out_v, out_hbm.at[pl.ds(obase + ci * CH, CH)], osems[b])
        for ci in range(nch):
            ocp[ci].wait()

    return k(logits, label, table)


def _tc_head(logits, label, table, btc):
    L, P = table.shape

    def body(x_ref, lab_ref, tab_ref, o_ref):
        lab = lab_ref[...]
        one_hot = (lab[:, None] ==
                   lax.broadcasted_iota(jnp.int32, (btc, L), 1))
        gathered = jnp.dot(one_hot.astype(jnp.float32), tab_ref[...],
                           preferred_element_type=jnp.float32)
        diff = x_ref[...] - gathered
        rowsum = jnp.sum(diff * diff, axis=1)
        o_ref[0, 0] = jnp.sum(jnp.sqrt(rowsum))

    out = pl.pallas_call(
        body,
        grid=(1,),
        in_specs=[
            pl.BlockSpec((btc, P), lambda i: (0, 0)),
            pl.BlockSpec((btc,), lambda i: (0,)),
            pl.BlockSpec((L, P), lambda i: (0, 0)),
        ],
        out_specs=pl.BlockSpec(memory_space=pltpu.SMEM),
        out_shape=jax.ShapeDtypeStruct((1, 1), jnp.float32),
    )(logits, label, table)
    return out


def _tc_finish(x, head_sum, B):
    def body(x_ref, h_ref, o_ref):
        o_ref[0, 0] = (jnp.sum(jnp.sqrt(x_ref[...])) + h_ref[0, 0]) / B

    out = pl.pallas_call(
        body,
        in_specs=[
            pl.BlockSpec(memory_space=pltpu.VMEM),
            pl.BlockSpec(memory_space=pltpu.SMEM),
        ],
        out_shape=jax.ShapeDtypeStruct((1, 1), jnp.float32),
        out_specs=pl.BlockSpec(memory_space=pltpu.SMEM),
    )(x, head_sum)
    return out[0, 0]


def kernel(logits, label, mean_expand):
    B = logits.shape[0]
    btc = B // 2
    bsc = B - btc
    label = label.astype(jnp.int32)
    head = _tc_head(logits, label, mean_expand, btc)
    part = _sc_partial_sumsq(logits, label, mean_expand, btc, bsc)
    return _tc_finish(part, head, B)


# speedup vs baseline: 1.1731x; 1.0305x over previous
import functools

import jax
import jax.numpy as jnp
from jax import lax
from jax.experimental import pallas as pl
from jax.experimental.pallas import tpu as pltpu
from jax.experimental.pallas import tpu_sc as plsc


def _sc_partial_sumsq(logits, label, table, row0, bsc):
    _, P = logits.shape
    L = table.shape[0]
    info = plsc.get_sparse_core_info()
    NC, NS, LN = info.num_cores, info.num_subcores, info.num_lanes
    NW = NC * NS
    rows_per = bsc // NW
    CH = rows_per // 2
    nch = rows_per // CH
    NB = nch

    mesh = plsc.VectorSubcoreMesh(core_axis_name="c", subcore_axis_name="s")

    @functools.partial(
        pl.kernel,
        mesh=mesh,
        out_type=jax.ShapeDtypeStruct((bsc,), jnp.float32),
        compiler_params=pltpu.CompilerParams(needs_layout_passes=False),
        scratch_types=[
            pltpu.VMEM((rows_per,), jnp.int32),
            pltpu.VMEM((L, P), jnp.float32),
            pltpu.VMEM((CH, P), jnp.float32),
            pltpu.VMEM((CH, P), jnp.float32),
            pltpu.VMEM((CH,), jnp.float32),
            pltpu.VMEM((CH,), jnp.float32),
            pltpu.SemaphoreType.DMA,
            pltpu.SemaphoreType.DMA,
            pltpu.SemaphoreType.DMA,
            pltpu.SemaphoreType.DMA,
        ],
    )
    def k(logits_hbm, label_hbm, table_hbm, out_hbm, idx_v, tab_v,
          log_v0, log_v1, out_v0, out_v1, lsem0, lsem1, osem0, osem1):
        wid = lax.axis_index("s") * NC + lax.axis_index("c")
        obase = pl.multiple_of(wid * rows_per, 8)
        base = row0 + obase
        logs = (log_v0, log_v1)
        outs = (out_v0, out_v1)
        lsems = (lsem0, lsem1)
        osems = (osem0, osem1)

        lcp = [pltpu.async_copy(
            logits_hbm.at[pl.ds(base + ci * CH, CH)], logs[ci], lsems[ci])
            for ci in range(nch)]
        pltpu.sync_copy(label_hbm.at[pl.ds(base, rows_per)], idx_v)
        pltpu.sync_copy(table_hbm, tab_v)

        iota = lax.iota(jnp.int32, LN)
        cols = [iota + j * LN for j in range(P // LN)]
        last = jnp.full((LN,), LN - 1, jnp.int32)
        z16 = jnp.zeros((LN,), jnp.float32)

        ocp = [None] * nch
        for ci in range(nch):
            b = ci % NB
            lcp[ci].wait()
            log_v = logs[b]
            out_v = outs[b]
            coff = ci * CH

            @plsc.parallel_loop(0, CH, 1, unroll=1, carry=z16)
            def row(r, sel):
                labv = plsc.load_gather(
                    idx_v, [jnp.full((LN,), coff + r, jnp.int32)])
                acc = jnp.zeros((LN,), jnp.float32)
                for j in range(P // LN):
                    t = plsc.load_gather(tab_v, [labv, cols[j]])
                    d = log_v[r, pl.ds(j * LN, LN)] - t
                    acc = acc + d * d
                cs = jnp.cumsum(acc)
                tot = jnp.take(cs, last)
                lane = r % LN
                sel = jnp.where(iota == lane, tot, sel)
                done = lane == LN - 1

                @pl.when(done)
                def _():
                    out_v[pl.ds((r // LN) * LN, LN)] = sel

                return jnp.where(done, z16, sel)

            ocp[ci] = pltpu.async_copy(
                out_v, out_hbm.at[pl.ds(obase + ci * CH, CH)], osems[b])
        for ci in range(nch):
            ocp[ci].wait()

    return k(logits, label, table)


def _tc_head(logits, label, table, btc):
    L, P = table.shape

    def body(x_ref, lab_ref, tab_ref, o_ref):
        lab = lab_ref[...]
        one_hot = (lab[:, None] ==
                   lax.broadcasted_iota(jnp.int32, (btc, L), 1))
        gathered = jnp.dot(one_hot.astype(jnp.float32), tab_ref[...],
                           preferred_element_type=jnp.float32)
        diff = x_ref[...] - gathered
        rowsum = jnp.sum(diff * diff, axis=1)
        o_ref[0, 0] = jnp.sum(jnp.sqrt(rowsum))

    out = pl.pallas_call(
        body,
        grid=(1,),
        in_specs=[
            pl.BlockSpec((btc, P), lambda i: (0, 0)),
            pl.BlockSpec((btc,), lambda i: (0,)),
            pl.BlockSpec((L, P), lambda i: (0, 0)),
        ],
        out_specs=pl.BlockSpec(memory_space=pltpu.SMEM),
        out_shape=jax.ShapeDtypeStruct((1, 1), jnp.float32),
    )(logits, label, table)
    return out


def _tc_finish(x, head_sum, B):
    def body(x_ref, h_ref, o_ref):
        o_ref[0, 0] = (jnp.sum(jnp.sqrt(x_ref[...])) + h_ref[0, 0]) / B

    out = pl.pallas_call(
        body,
        in_specs=[
            pl.BlockSpec(memory_space=pltpu.VMEM),
            pl.BlockSpec(memory_space=pltpu.SMEM),
        ],
        out_shape=jax.ShapeDtypeStruct((1, 1), jnp.float32),
        out_specs=pl.BlockSpec(memory_space=pltpu.SMEM),
    )(x, head_sum)
    return out[0, 0]


def kernel(logits, label, mean_expand):
    B = logits.shape[0]
    btc = 5 * B // 8
    bsc = B - btc
    label = label.astype(jnp.int32)
    head = _tc_head(logits, label, mean_expand, btc)
    part = _sc_partial_sumsq(logits, label, mean_expand, btc, bsc)
    return _tc_finish(part, head, B)


# speedup vs baseline: 1.2123x; 1.0334x over previous
import functools

import jax
import jax.numpy as jnp
from jax import lax
from jax.experimental import pallas as pl
from jax.experimental.pallas import tpu as pltpu
from jax.experimental.pallas import tpu_sc as plsc


def _sc_partial_sumsq(logits, label, table, row0, bsc):
    _, P = logits.shape
    L = table.shape[0]
    info = plsc.get_sparse_core_info()
    NC, NS, LN = info.num_cores, info.num_subcores, info.num_lanes
    NW = NC * NS
    rows_per = bsc // NW
    CH = rows_per // 2
    nch = rows_per // CH
    NB = nch

    mesh = plsc.VectorSubcoreMesh(core_axis_name="c", subcore_axis_name="s")

    @functools.partial(
        pl.kernel,
        mesh=mesh,
        out_type=jax.ShapeDtypeStruct((bsc,), jnp.float32),
        compiler_params=pltpu.CompilerParams(needs_layout_passes=False),
        scratch_types=[
            pltpu.VMEM((rows_per,), jnp.int32),
            pltpu.VMEM((L, P), jnp.float32),
            pltpu.VMEM((CH, P), jnp.float32),
            pltpu.VMEM((CH, P), jnp.float32),
            pltpu.VMEM((CH,), jnp.float32),
            pltpu.VMEM((CH,), jnp.float32),
            pltpu.SemaphoreType.DMA,
            pltpu.SemaphoreType.DMA,
            pltpu.SemaphoreType.DMA,
            pltpu.SemaphoreType.DMA,
        ],
    )
    def k(logits_hbm, label_hbm, table_hbm, out_hbm, idx_v, tab_v,
          log_v0, log_v1, out_v0, out_v1, lsem0, lsem1, osem0, osem1):
        wid = lax.axis_index("s") * NC + lax.axis_index("c")
        obase = pl.multiple_of(wid * rows_per, 8)
        base = row0 + obase
        logs = (log_v0, log_v1)
        outs = (out_v0, out_v1)
        lsems = (lsem0, lsem1)
        osems = (osem0, osem1)

        lcp = [pltpu.async_copy(
            logits_hbm.at[pl.ds(base + ci * CH, CH)], logs[ci], lsems[ci])
            for ci in range(nch)]
        pltpu.sync_copy(label_hbm.at[pl.ds(base, rows_per)], idx_v)
        pltpu.sync_copy(table_hbm, tab_v)

        iota = lax.iota(jnp.int32, LN)
        cols = [iota + j * LN for j in range(P // LN)]
        last = jnp.full((LN,), LN - 1, jnp.int32)
        z16 = jnp.zeros((LN,), jnp.float32)

        ocp = [None] * nch
        for ci in range(nch):
            b = ci % NB
            lcp[ci].wait()
            log_v = logs[b]
            out_v = outs[b]
            coff = ci * CH

            @plsc.parallel_loop(0, CH, 1, unroll=1, carry=z16)
            def row(r, sel):
                labv = plsc.load_gather(
                    idx_v, [jnp.full((LN,), coff + r, jnp.int32)])
                acc = jnp.zeros((LN,), jnp.float32)
                for j in range(P // LN):
                    t = plsc.load_gather(tab_v, [labv, cols[j]])
                    d = log_v[r, pl.ds(j * LN, LN)] - t
                    acc = acc + d * d
                cs = jnp.cumsum(acc)
                tot = jnp.take(cs, last)
                lane = r % LN
                sel = jnp.where(iota == lane, tot, sel)
                done = lane == LN - 1

                @pl.when(done)
                def _():
                    out_v[pl.ds((r // LN) * LN, LN)] = sel

                return jnp.where(done, z16, sel)

            ocp[ci] = pltpu.async_copy(
                out_v, out_hbm.at[pl.ds(obase + ci * CH, CH)], osems[b])
        for ci in range(nch):
            ocp[ci].wait()

    return k(logits, label, table)


def _tc_head(logits, label, table, btc):
    L, P = table.shape

    def body(x_ref, lab_ref, tab_ref, o_ref):
        lab = lab_ref[...]
        one_hot = (lab[:, None] ==
                   lax.broadcasted_iota(jnp.int32, (btc, L), 1))
        gathered = jnp.dot(one_hot.astype(jnp.float32), tab_ref[...],
                           preferred_element_type=jnp.float32)
        diff = x_ref[...] - gathered
        rowsum = jnp.sum(diff * diff, axis=1)
        o_ref[0, 0] = jnp.sum(jnp.sqrt(rowsum))

    out = pl.pallas_call(
        body,
        grid=(1,),
        in_specs=[
            pl.BlockSpec((btc, P), lambda i: (0, 0)),
            pl.BlockSpec((btc,), lambda i: (0,)),
            pl.BlockSpec((L, P), lambda i: (0, 0)),
        ],
        out_specs=pl.BlockSpec(memory_space=pltpu.SMEM),
        out_shape=jax.ShapeDtypeStruct((1, 1), jnp.float32),
    )(logits, label, table)
    return out


def _tc_finish(x, head_sum, B):
    def body(x_ref, h_ref, o_ref):
        o_ref[0, 0] = (jnp.sum(jnp.sqrt(x_ref[...])) + h_ref[0, 0]) / B

    out = pl.pallas_call(
        body,
        in_specs=[
            pl.BlockSpec(memory_space=pltpu.VMEM),
            pl.BlockSpec(memory_space=pltpu.SMEM),
        ],
        out_shape=jax.ShapeDtypeStruct((1, 1), jnp.float32),
        out_specs=pl.BlockSpec(memory_space=pltpu.SMEM),
    )(x, head_sum)
    return out[0, 0]


def kernel(logits, label, mean_expand):
    B = logits.shape[0]
    btc = 3 * B // 4
    bsc = B - btc
    label = label.astype(jnp.int32)
    head = _tc_head(logits, label, mean_expand, btc)
    part = _sc_partial_sumsq(logits, label, mean_expand, btc, bsc)
    return _tc_finish(part, head, B)
